# Initial kernel scaffold; baseline (speedup 1.0000x reference)
import jax
import jax.numpy as jnp
from jax.experimental import pallas as pl


def kernel(x, h, edge_index, edge_fea, v, loc_mean, timeframes, params):
    raise NotImplementedError("write your pallas kernel here")


# speedup vs baseline: 1.1495x; 1.1495x over previous
import functools
import jax
import jax.numpy as jnp
from jax import lax
from jax.experimental import pallas as pl
from jax.experimental.pallas import tpu as pltpu
from jax.experimental.pallas import tpu_sc as plsc

N_NODES = 10000
N_EDGES = 40000
N_ATOMS = 5
T = 8
B = N_NODES // N_ATOMS
IN_NODE_NF = 16
HID_ENC = 64
TIME_EMB_DIM = 32
HID_DEC = 96
DELTA_FRAME = 1.0
XPAD = 16
MSG_ENC = 80
MSG_DEC = 112
BLK = 2000


def _silu(z):
    return z * jax.nn.sigmoid(z)


def _edge_body(hpre, xij, ef, wd2, wef, be1, we2, be2, wx1, bx1, wx2, bx2,
               out):
    x = xij[...]
    d2 = jnp.sum(x * x, axis=1, keepdims=True)
    m = _silu(hpre[...] + d2 * wd2[...] + ef[...] @ wef[...] + be1[...])
    m = _silu(m @ we2[...] + be2[...])
    u = _silu(m @ wx1[...] + bx1[...])
    w = u @ wx2[...] + bx2[...]
    blk = x.shape[0]
    ones = jnp.ones((blk, 1), jnp.float32)
    pad = out.shape[1] - (m.shape[1] + 4 + 1)
    out[...] = jnp.concatenate(
        [m, x[:, :4] * w, ones, jnp.zeros((blk, pad), jnp.float32)], axis=1)


def _edge_mlp(hpre, xij, ef, p, hid, msg_w):
    E = hpre.shape[0]
    we1 = p["We1"]
    wd2 = we1[2 * hid:2 * hid + 1]
    wef = we1[2 * hid + 1:]
    grid = E // BLK
    row = lambda i: (i, 0)
    full = lambda *_: (0, 0)
    return pl.pallas_call(
        _edge_body,
        grid=(grid,),
        in_specs=[
            pl.BlockSpec((BLK, hid), row),
            pl.BlockSpec((BLK, XPAD), row),
            pl.BlockSpec((BLK, 4), row),
            pl.BlockSpec((1, hid), full),
            pl.BlockSpec((4, hid), full),
            pl.BlockSpec((1, hid), full),
            pl.BlockSpec((hid, hid), full),
            pl.BlockSpec((1, hid), full),
            pl.BlockSpec((hid, hid), full),
            pl.BlockSpec((1, hid), full),
            pl.BlockSpec((hid, 1), full),
            pl.BlockSpec((1, 1), full),
        ],
        out_specs=pl.BlockSpec((BLK, msg_w), row),
        out_shape=jax.ShapeDtypeStruct((E, msg_w), jnp.float32),
    )(hpre, xij, ef, wd2, wef, p["be1"][None], p["We2"], p["be2"][None],
      p["Wx1"], p["bx1"][None], p["Wx2"], p["bx2"][None])


def _node_body(emit_yz, hid, x4, h, agg, wh1a, wh1b, bh1, wh2, bh2, wa, wb,
               x4o, ho, yo, zo):
    a = agg[...]
    agg_h = a[:, :hid]
    cnt = jnp.maximum(a[:, hid + 4:hid + 5], 1.0)
    blk = a.shape[0]
    xupd = jnp.concatenate(
        [a[:, hid:hid + 4], jnp.zeros((blk, XPAD - 4), jnp.float32)], axis=1)
    x4o[...] = x4[...] + xupd / cnt
    hh = h[...]
    upd = _silu(hh @ wh1a[...] + agg_h @ wh1b[...] + bh1[...])
    hn = hh + upd @ wh2[...] + bh2[...]
    ho[...] = hn
    if emit_yz:
        yo[...] = hn @ wa[...]
        zo[...] = hn @ wb[...]


def _node_update(x4, h, agg, p, hid, msg_w, nxt=None):
    M = h.shape[0]
    emit_yz = nxt is not None
    nhid = nxt.shape[1] if emit_yz else hid
    wa = nxt[:hid] if emit_yz else jnp.zeros((hid, hid), jnp.float32)
    wb = nxt[hid:2 * hid] if emit_yz else jnp.zeros((hid, hid), jnp.float32)
    grid = M // BLK
    row = lambda i: (i, 0)
    full = lambda *_: (0, 0)
    outs = [jax.ShapeDtypeStruct((M, XPAD), jnp.float32),
            jax.ShapeDtypeStruct((M, hid), jnp.float32),
            jax.ShapeDtypeStruct((M, nhid), jnp.float32),
            jax.ShapeDtypeStruct((M, nhid), jnp.float32)]
    res = pl.pallas_call(
        functools.partial(_node_body, emit_yz, hid),
        grid=(grid,),
        in_specs=[
            pl.BlockSpec((BLK, XPAD), row),
            pl.BlockSpec((BLK, hid), row),
            pl.BlockSpec((BLK, msg_w), row),
            pl.BlockSpec((hid, hid), full),
            pl.BlockSpec((hid, hid), full),
            pl.BlockSpec((1, hid), full),
            pl.BlockSpec((hid, hid), full),
            pl.BlockSpec((1, hid), full),
            pl.BlockSpec((hid, nhid), full),
            pl.BlockSpec((hid, nhid), full),
        ],
        out_specs=[pl.BlockSpec((BLK, XPAD), row),
                   pl.BlockSpec((BLK, hid), row),
                   pl.BlockSpec((BLK, nhid), row),
                   pl.BlockSpec((BLK, nhid), row)],
        out_shape=outs,
    )(x4, h, agg, p["Wh1"][:hid], p["Wh1"][hid:], p["bh1"][None], p["Wh2"],
      p["bh2"][None], wa, wb)
    if emit_yz:
        return res[0], res[1], res[2], res[3]
    return res[0], res[1]


def _embed_body(h0, wemb, bemb, wa, wb, wt, ho, yo, zo, dho):
    hh = h0[...] @ wemb[...] + bemb[...]
    ho[...] = hh
    yo[...] = hh @ wa[...]
    zo[...] = hh @ wb[...]
    dho[...] = _silu(hh @ wt[...])


def _embed(h0, params):
    N = h0.shape[0]
    hid = HID_ENC
    we1 = params["enc0"]["We1"]
    grid = N // BLK
    row = lambda i: (i, 0)
    full = lambda *_: (0, 0)
    return pl.pallas_call(
        _embed_body,
        grid=(grid,),
        in_specs=[
            pl.BlockSpec((BLK, IN_NODE_NF), row),
            pl.BlockSpec((IN_NODE_NF, hid), full),
            pl.BlockSpec((1, hid), full),
            pl.BlockSpec((hid, hid), full),
            pl.BlockSpec((hid, hid), full),
            pl.BlockSpec((hid, hid), full),
        ],
        out_specs=[pl.BlockSpec((BLK, hid), row)] * 4,
        out_shape=[jax.ShapeDtypeStruct((N, hid), jnp.float32)] * 4,
    )(h0, params["W_emb"], params["b_emb"][None], we1[:hid], we1[hid:2 * hid],
      params["W_t"])


def _time_body(wxp, freqs, wa, wb, h, dh, x4, v4, lm, tfc,
               hho, x4o, v4o, yo, zo):
    w = wxp[...]
    a, bb, c, d = w[:, 0:1], w[:, 1:2], w[:, 2:3], w[:, 3:4]
    tf = tfc[:, :1]
    xtr = x4[...] - lm[...]
    vv = v4[...]
    xo = xtr + tf * (a * xtr + c * vv) + lm[...]
    vo = vv + tf * (bb * xtr + d * vv)
    x4o[...] = xo
    v4o[...] = vo
    ht = h[...] + tf * dh[...]
    args = tf * freqs[...]
    emb = jnp.concatenate([jnp.sin(args), jnp.cos(args)], axis=1)
    hho[...] = jnp.concatenate([ht, emb], axis=1)
    yo[...] = ht @ wa[:HID_ENC] + emb @ wa[HID_ENC:]
    zo[...] = ht @ wb[:HID_ENC] + emb @ wb[HID_ENC:]


def _time_expand(h_rep, dh_rep, x4_rep, v4_rep, lm_rep, tf_col, params):
    M = h_rep.shape[0]
    we1 = params["dec0"]["We1"]
    wa = we1[:HID_DEC]
    wb = we1[HID_DEC:2 * HID_DEC]
    import numpy as np
    half = TIME_EMB_DIM // 2
    scale = np.log(10000.0) / (half - 1)
    freqs = jnp.exp(-scale * jnp.arange(half, dtype=jnp.float32))[None]
    wxp = params["W_x"].reshape(1, 4)
    grid = M // BLK
    row = lambda i: (i, 0)
    full = lambda *_: (0, 0)
    return pl.pallas_call(
        _time_body,
        grid=(grid,),
        in_specs=[
            pl.BlockSpec((1, 4), full),
            pl.BlockSpec((1, half), full),
            pl.BlockSpec((HID_DEC, HID_DEC), full),
            pl.BlockSpec((HID_DEC, HID_DEC), full),
            pl.BlockSpec((BLK, HID_ENC), row),
            pl.BlockSpec((BLK, HID_ENC), row),
            pl.BlockSpec((BLK, XPAD), row),
            pl.BlockSpec((BLK, XPAD), row),
            pl.BlockSpec((BLK, XPAD), row),
            pl.BlockSpec((BLK, 8), row),
        ],
        out_specs=[pl.BlockSpec((BLK, HID_DEC), row),
                   pl.BlockSpec((BLK, XPAD), row),
                   pl.BlockSpec((BLK, XPAD), row),
                   pl.BlockSpec((BLK, HID_DEC), row),
                   pl.BlockSpec((BLK, HID_DEC), row)],
        out_shape=[jax.ShapeDtypeStruct((M, HID_DEC), jnp.float32),
                   jax.ShapeDtypeStruct((M, XPAD), jnp.float32),
                   jax.ShapeDtypeStruct((M, XPAD), jnp.float32),
                   jax.ShapeDtypeStruct((M, HID_DEC), jnp.float32),
                   jax.ShapeDtypeStruct((M, HID_DEC), jnp.float32)],
    )(wxp, freqs, wa, wb, h_rep, dh_rep, x4_rep, v4_rep, lm_rep, tf_col)


def _gather_stage(y, z, x4, x4n, e0, e1):
    hpre = y[e0] + z[e1]
    xij = x4[e0] + x4n[e1]
    return hpre, xij


def _scatter_stage(msg, e0, nseg):
    return jax.ops.segment_sum(msg, e0, num_segments=nseg)


def _pad4(x):
    return jnp.pad(x, ((0, 0), (0, XPAD - x.shape[1])))


def kernel(x, h, edge_index, edge_fea, v, loc_mean, timeframes, params):
    e0 = edge_index[0]
    e1 = edge_index[1]

    x4 = _pad4(x)
    h1, y, z, dh = _embed(h, params)
    for i in range(2):
        p = params["enc%d" % i]
        hpre, xij = _gather_stage(y, z, x4, -x4, e0, e1)
        msg = _edge_mlp(hpre, xij, edge_fea, p, HID_ENC, MSG_ENC)
        agg = _scatter_stage(msg, e0, N_NODES)
        if i == 0:
            nx = params["enc1"]["We1"]
            x4, h1, y, z = _node_update(x4, h1, agg, p, HID_ENC, MSG_ENC, nx)
        else:
            x4, h1 = _node_update(x4, h1, agg, p, HID_ENC, MSG_ENC)

    dh = _dh_only(h1, params)

    tf_t = (timeframes / DELTA_FRAME).T
    tf_node = jnp.repeat(tf_t.reshape(-1), N_ATOMS)
    tf_col = jnp.broadcast_to(tf_node[:, None], (T * N_NODES, 8))
    tile = lambda a: jnp.tile(a, (T, 1))
    hh, x4d, v4d, y, z = _time_expand(
        tile(h1), tile(dh), tile(x4), tile(_pad4(v)), tile(_pad4(loc_mean)),
        tf_col, params)

    offs = jnp.repeat(jnp.arange(T, dtype=e0.dtype) * N_NODES, N_EDGES)
    E0 = jnp.tile(e0, (T,)) + offs
    E1 = jnp.tile(e1, (T,)) + offs
    EF = jnp.tile(edge_fea, (T, 1))
    for i in range(2):
        p = params["dec%d" % i]
        hpre, xij = _gather_stage(y, z, x4d, -x4d, E0, E1)
        msg = _edge_mlp(hpre, xij, EF, p, HID_DEC, MSG_DEC)
        agg = _scatter_stage(msg, E0, T * N_NODES)
        if i == 0:
            nx = params["dec1"]["We1"]
            x4d, hh, y, z = _node_update(x4d, hh, agg, p, HID_DEC, MSG_DEC, nx)
        else:
            x4d, hh = _node_update(x4d, hh, agg, p, HID_DEC, MSG_DEC)

    xx = x4d[:, :3]
    vv = v4d[:, :3]
    kld = jnp.asarray(0.0, jnp.float32)
    return (xx, vv, hh, kld)


def _dh_body(h, wt, dho):
    dho[...] = _silu(h[...] @ wt[...])


def _dh_only(h1, params):
    N = h1.shape[0]
    grid = N // BLK
    row = lambda i: (i, 0)
    full = lambda *_: (0, 0)
    return pl.pallas_call(
        _dh_body,
        grid=(grid,),
        in_specs=[pl.BlockSpec((BLK, HID_ENC), row),
                  pl.BlockSpec((HID_ENC, HID_ENC), full)],
        out_specs=pl.BlockSpec((BLK, HID_ENC), row),
        out_shape=jax.ShapeDtypeStruct((N, HID_ENC), jnp.float32),
    )(h1, params["W_t"])


# speedup vs baseline: 2.5614x; 2.2283x over previous
import functools
import jax
import jax.numpy as jnp
from jax import lax
from jax.experimental import pallas as pl
from jax.experimental.pallas import tpu as pltpu
from jax.experimental.pallas import tpu_sc as plsc

N_NODES = 10000
N_EDGES = 40000
N_ATOMS = 5
T = 8
B = N_NODES // N_ATOMS
IN_NODE_NF = 16
HID_ENC = 64
TIME_EMB_DIM = 32
HID_DEC = 96
DELTA_FRAME = 1.0
XPAD = 16
MSG_ENC = 128
MSG_DEC = 128
BLK = 2000
BLK_E = 2048
EP = 40960
NROW = 10112
NW = 32
GB = 128
CS = 640
CATW = 256


def _silu(z):
    return z * jax.nn.sigmoid(z)


def _mm(a, b):
    return jax.lax.dot_general(a, b, (((1,), (0,)), ((), ())),
                               precision=jax.lax.Precision.DEFAULT,
                               preferred_element_type=jnp.float32)


def _edge_body(hid, hi, hj, ef, we1, be1, we2, be2, wx1, bx1, wx2, bx2,
               out):
    a = hi[...]
    b = hj[...]
    x = a[:, hid:hid + XPAD] - b[:, hid:hid + XPAD]
    d2 = jnp.sum(x * x, axis=1, keepdims=True)
    blk = a.shape[0]
    catpad = jnp.zeros((blk, CATW - 2 * hid - 1 - 4), jnp.float32)
    cat = jnp.concatenate([a[:, :hid], b[:, :hid], d2, ef[...], catpad],
                          axis=1)
    m = _silu(_mm(cat, we1[...]) + be1[...])
    m = _silu(_mm(m, we2[...]) + be2[...])
    u = _silu(_mm(m, wx1[...]) + bx1[...])
    w = _mm(u, wx2[...]) + bx2[...]
    blk = a.shape[0]
    ones = jnp.ones((blk, 1), jnp.float32)
    pad = out.shape[1] - (m.shape[1] + 4 + 1)
    out[...] = jnp.concatenate(
        [m, x[:, :4] * w, ones, jnp.zeros((blk, pad), jnp.float32)], axis=1)


def _edge_mlp(hi, hj, ef, p, hid, msg_w):
    E = hi.shape[0]
    we1 = jnp.pad(p["We1"], ((0, CATW - p["We1"].shape[0]), (0, 0)))
    grid = E // BLK_E
    row = lambda i: (i, 0)
    full = lambda *_: (0, 0)
    return pl.pallas_call(
        functools.partial(_edge_body, hid),
        grid=(grid,),
        in_specs=[
            pl.BlockSpec((BLK_E, 128), row),
            pl.BlockSpec((BLK_E, 128), row),
            pl.BlockSpec((BLK_E, 4), row),
            pl.BlockSpec((CATW, hid), full),
            pl.BlockSpec((1, hid), full),
            pl.BlockSpec((hid, hid), full),
            pl.BlockSpec((1, hid), full),
            pl.BlockSpec((hid, hid), full),
            pl.BlockSpec((1, hid), full),
            pl.BlockSpec((hid, 1), full),
            pl.BlockSpec((1, 1), full),
        ],
        out_specs=pl.BlockSpec((BLK_E, msg_w), row),
        out_shape=jax.ShapeDtypeStruct((E, msg_w), jnp.float32),
    )(hi, hj, ef, we1, p["be1"][None], p["We2"], p["be2"][None],
      p["Wx1"], p["bx1"][None], p["Wx2"], p["bx2"][None])


def _node_body(emit_tab, hid, x4, h, agg, wh1, bh1, wh2, bh2,
               x4o, ho, tabo):
    a = agg[...]
    agg_h = a[:, :hid]
    cnt = jnp.maximum(a[:, hid + 4:hid + 5], 1.0)
    blk = a.shape[0]
    xupd = jnp.concatenate(
        [a[:, hid:hid + 4], jnp.zeros((blk, XPAD - 4), jnp.float32)], axis=1)
    x4n = x4[...] + xupd / cnt
    x4o[...] = x4n
    hh = h[...]
    cat = jnp.concatenate([hh, agg_h], axis=1)
    upd = _silu(_mm(cat, wh1[...]) + bh1[...])
    hn = hh + _mm(upd, wh2[...]) + bh2[...]
    ho[...] = hn
    if emit_tab:
        pad = jnp.zeros((blk, 128 - hn.shape[1] - XPAD), jnp.float32)
        tabo[...] = jnp.concatenate([hn, x4n, pad], axis=1)


def _node_update(x4, h, agg, p, hid, msg_w, emit_tab=False):
    M = h.shape[0]
    grid = M // BLK
    row = lambda i: (i, 0)
    full = lambda *_: (0, 0)
    outs = [jax.ShapeDtypeStruct((M, XPAD), jnp.float32),
            jax.ShapeDtypeStruct((M, hid), jnp.float32),
            jax.ShapeDtypeStruct((M, 128), jnp.float32)]
    res = pl.pallas_call(
        functools.partial(_node_body, emit_tab, hid),
        grid=(grid,),
        in_specs=[
            pl.BlockSpec((BLK, XPAD), row),
            pl.BlockSpec((BLK, hid), row),
            pl.BlockSpec((BLK, msg_w), row),
            pl.BlockSpec((2 * hid, hid), full),
            pl.BlockSpec((1, hid), full),
            pl.BlockSpec((hid, hid), full),
            pl.BlockSpec((1, hid), full),
        ],
        out_specs=[pl.BlockSpec((BLK, XPAD), row),
                   pl.BlockSpec((BLK, hid), row),
                   pl.BlockSpec((BLK, 128), row)],
        out_shape=outs,
    )(x4, h, agg, p["Wh1"], p["bh1"][None], p["Wh2"], p["bh2"][None])
    if emit_tab:
        return res[0], res[1], res[2]
    return res[0], res[1]


def _embed_body(h0, x4, wemb, bemb, ho, tabo):
    hh = _mm(h0[...], wemb[...]) + bemb[...]
    xx = x4[...]
    ho[...] = hh
    pad = jnp.zeros((hh.shape[0], 128 - hh.shape[1] - XPAD), jnp.float32)
    tabo[...] = jnp.concatenate([hh, xx, pad], axis=1)


def _embed(h0, x4, params):
    N = h0.shape[0]
    hid = HID_ENC
    grid = N // BLK
    row = lambda i: (i, 0)
    full = lambda *_: (0, 0)
    return pl.pallas_call(
        _embed_body,
        grid=(grid,),
        in_specs=[
            pl.BlockSpec((BLK, IN_NODE_NF), row),
            pl.BlockSpec((BLK, XPAD), row),
            pl.BlockSpec((IN_NODE_NF, hid), full),
            pl.BlockSpec((1, hid), full),
        ],
        out_specs=[pl.BlockSpec((BLK, hid), row),
                   pl.BlockSpec((BLK, 128), row)],
        out_shape=[jax.ShapeDtypeStruct((N, hid), jnp.float32),
                   jax.ShapeDtypeStruct((N, 128), jnp.float32)],
    )(h0, x4, params["W_emb"], params["b_emb"][None])


def _time_body(wxp, freqs, h, dh, x4, v4, lm, tfc,
               hho, x4o, v4o, tabo):
    w = wxp[...]
    a, bb, c, d = w[:, 0:1], w[:, 1:2], w[:, 2:3], w[:, 3:4]
    tf = tfc[:, :1]
    xtr = x4[...] - lm[...]
    vv = v4[...]
    xo = xtr + tf * (a * xtr + c * vv) + lm[...]
    vo = vv + tf * (bb * xtr + d * vv)
    x4o[...] = xo
    v4o[...] = vo
    ht = h[...] + tf * dh[...]
    args = tf * freqs[...]
    emb = jnp.concatenate([jnp.sin(args), jnp.cos(args)], axis=1)
    hhcat = jnp.concatenate([ht, emb], axis=1)
    hho[...] = hhcat
    pad = jnp.zeros((ht.shape[0], 128 - HID_DEC - XPAD), jnp.float32)
    tabo[...] = jnp.concatenate([hhcat, xo, pad], axis=1)


def _time_expand(h_rep, dh_rep, x4_rep, v4_rep, lm_rep, tf_col, params):
    M = h_rep.shape[0]
    import numpy as np
    half = TIME_EMB_DIM // 2
    scale = np.log(10000.0) / (half - 1)
    freqs = jnp.exp(-scale * jnp.arange(half, dtype=jnp.float32))[None]
    wxp = params["W_x"].reshape(1, 4)
    grid = M // BLK
    row = lambda i: (i, 0)
    full = lambda *_: (0, 0)
    return pl.pallas_call(
        _time_body,
        grid=(grid,),
        in_specs=[
            pl.BlockSpec((1, 4), full),
            pl.BlockSpec((1, half), full),
            pl.BlockSpec((BLK, HID_ENC), row),
            pl.BlockSpec((BLK, HID_ENC), row),
            pl.BlockSpec((BLK, XPAD), row),
            pl.BlockSpec((BLK, XPAD), row),
            pl.BlockSpec((BLK, XPAD), row),
            pl.BlockSpec((BLK, 8), row),
        ],
        out_specs=[pl.BlockSpec((BLK, HID_DEC), row),
                   pl.BlockSpec((BLK, XPAD), row),
                   pl.BlockSpec((BLK, XPAD), row),
                   pl.BlockSpec((BLK, 128), row)],
        out_shape=[jax.ShapeDtypeStruct((M, HID_DEC), jnp.float32),
                   jax.ShapeDtypeStruct((M, XPAD), jnp.float32),
                   jax.ShapeDtypeStruct((M, XPAD), jnp.float32),
                   jax.ShapeDtypeStruct((M, 128), jnp.float32)],
    )(wxp, freqs, h_rep, dh_rep, x4_rep, v4_rep, lm_rep, tf_col)


def _sc_gather(tab, idx0, idx1, E, D):
    C = E // (NW * CS)
    R = E // (NW * GB)
    K = CS // GB
    mesh = plsc.VectorSubcoreMesh(core_axis_name="c", subcore_axis_name="s", num_cores=2, num_subcores=16)

    def body(tab_h, i0_h, i1_h, hi_h, hj_h, iv0, iv1, buf):
        w = lax.axis_index("s") * 2 + lax.axis_index("c")
        pltpu.sync_copy(i0_h.at[w], iv0)
        pltpu.sync_copy(i1_h.at[w], iv1)

        def chunk(tab, iv, out, c):
            for j in range(K):
                pltpu.sync_copy(tab.at[iv.at[c * K + j]],
                                buf.at[pl.ds(j * GB, GB)])
            pltpu.sync_copy(buf, out.at[pl.ds(w * C * CS + c * CS, CS)])

        lax.fori_loop(0, C, lambda c, _: (chunk(tab_h, iv0, hi_h, c), 0)[1], 0)
        lax.fori_loop(0, C, lambda c, _: (chunk(tab_h, iv1, hj_h, c), 0)[1], 0)

    f = pl.kernel(
        body,
        out_type=[jax.ShapeDtypeStruct((E, D), jnp.float32),
                  jax.ShapeDtypeStruct((E, D), jnp.float32)],
        mesh=mesh,
        scratch_types=[pltpu.VMEM((R, GB), jnp.int32),
                       pltpu.VMEM((R, GB), jnp.int32),
                       pltpu.VMEM((CS, D), jnp.float32)],
    )
    return f(tab, idx0, idx1)


def _sc_scatter(msg, idx, zrows, nt, D):
    SCS = 128
    ET = EP // 16
    CT = ET // SCS
    K = SCS // GB
    RT = EP // GB // 16
    WR = NROW // 16
    mesh = plsc.VectorSubcoreMesh(core_axis_name="c", subcore_axis_name="s", num_cores=2, num_subcores=16)

    def body(msg_h, idx_h, z_h, agg_h, iv, mbuf, acc):
        cc = lax.axis_index("c")
        s = lax.axis_index("s")
        pltpu.sync_copy(idx_h.at[s], iv)
        ncopies = msg_h.shape[0] // EP
        for k in range(nt):
            t = cc * nt + k

            @pl.when(t < ncopies)
            def _per_copy():
                pltpu.sync_copy(z_h, acc.at[pl.ds(s * WR, WR)])
                plsc.subcore_barrier()

                def chunk(c, _):
                    base = t * EP + s * ET + c * SCS
                    pltpu.sync_copy(msg_h.at[pl.ds(base, SCS)], mbuf)
                    for j in range(K):
                        pltpu.sync_copy(mbuf.at[pl.ds(j * GB, GB)],
                                        acc.at[iv.at[c * K + j]], add=True)
                    return 0

                lax.fori_loop(0, CT, chunk, 0)
                plsc.subcore_barrier()

                @pl.when(s < 15)
                def _():
                    pltpu.sync_copy(acc.at[pl.ds(s * WR, WR)],
                                    agg_h.at[pl.ds(t * N_NODES + s * WR, WR)])

                @pl.when(s == 15)
                def _():
                    pltpu.sync_copy(
                        acc.at[pl.ds(15 * WR, N_NODES - 15 * WR)],
                        agg_h.at[pl.ds(t * N_NODES + 15 * WR,
                                       N_NODES - 15 * WR)])

    ncopies = msg.shape[0] // EP
    f = pl.kernel(
        body,
        out_type=jax.ShapeDtypeStruct((ncopies * N_NODES, D), jnp.float32),
        mesh=mesh,
        scratch_types=[pltpu.VMEM((RT, GB), jnp.int32),
                       pltpu.VMEM((SCS, D), jnp.float32),
                       pltpu.VMEM_SHARED((NROW, D), jnp.float32)],
    )
    return f(msg, idx, zrows)


def _pad4(x):
    return jnp.pad(x, ((0, 0), (0, XPAD - x.shape[1])))


def kernel(x, h, edge_index, edge_fea, v, loc_mean, timeframes, params):
    e0 = edge_index[0]
    e1 = edge_index[1]
    npad = EP - N_EDGES

    e0p = jnp.concatenate([e0, jnp.zeros((npad,), e0.dtype)])
    e1p = jnp.concatenate([e1, jnp.zeros((npad,), e1.dtype)])
    scat_idx = jnp.concatenate(
        [e0, jnp.full((npad,), N_NODES, e0.dtype)]).reshape(16, EP // GB // 16, GB)
    offs = jnp.repeat(jnp.arange(T, dtype=e0.dtype) * N_NODES, EP)
    rdec = T * EP // (NW * GB)
    renc = EP // (NW * GB)
    idx0_dec = (jnp.tile(e0p, (T,)) + offs).reshape(NW, rdec, GB)
    idx1_dec = (jnp.tile(e1p, (T,)) + offs).reshape(NW, rdec, GB)
    idx0_enc = e0p.reshape(NW, renc, GB)
    idx1_enc = e1p.reshape(NW, renc, GB)
    efp = jnp.concatenate([edge_fea, jnp.zeros((npad, 4), jnp.float32)])
    ef_dec = jnp.tile(efp, (T, 1))
    zrows_enc = jnp.zeros((NROW // 16, MSG_ENC), jnp.float32)
    zrows_dec = jnp.zeros((NROW // 16, MSG_DEC), jnp.float32)

    x4 = _pad4(x)
    h1, tab = _embed(h, x4, params)
    for i in range(2):
        p = params["enc%d" % i]
        hi, hj = _sc_gather(tab, idx0_enc, idx1_enc, EP, 128)
        msg = _edge_mlp(hi, hj, efp, p, HID_ENC, MSG_ENC)
        agg = _sc_scatter(msg, scat_idx, zrows_enc, 1, MSG_ENC)
        if i == 0:
            x4, h1, tab = _node_update(x4, h1, agg, p, HID_ENC, MSG_ENC, True)
        else:
            x4, h1 = _node_update(x4, h1, agg, p, HID_ENC, MSG_ENC)

    dh = _dh_only(h1, params)

    tf_t = (timeframes / DELTA_FRAME).T
    tf_node = jnp.repeat(tf_t.reshape(-1), N_ATOMS)
    tf_col = jnp.broadcast_to(tf_node[:, None], (T * N_NODES, 8))
    tile = lambda a: jnp.tile(a, (T, 1))
    hh, x4d, v4d, tab = _time_expand(
        tile(h1), tile(dh), tile(x4), tile(_pad4(v)), tile(_pad4(loc_mean)),
        tf_col, params)

    for i in range(2):
        p = params["dec%d" % i]
        hi, hj = _sc_gather(tab, idx0_dec, idx1_dec, T * EP, 128)
        msg = _edge_mlp(hi, hj, ef_dec, p, HID_DEC, MSG_DEC)
        agg = _sc_scatter(msg, scat_idx, zrows_dec, 4, MSG_DEC)
        if i == 0:
            x4d, hh, tab = _node_update(x4d, hh, agg, p, HID_DEC, MSG_DEC,
                                        True)
        else:
            x4d, hh = _node_update(x4d, hh, agg, p, HID_DEC, MSG_DEC)

    xx = x4d[:, :3]
    vv = v4d[:, :3]
    kld = jnp.asarray(0.0, jnp.float32)
    return (xx, vv, hh, kld)


def _dh_body(h, wt, dho):
    dho[...] = _silu(_mm(h[...], wt[...]))


def _dh_only(h1, params):
    N = h1.shape[0]
    grid = N // BLK
    row = lambda i: (i, 0)
    full = lambda *_: (0, 0)
    return pl.pallas_call(
        _dh_body,
        grid=(grid,),
        in_specs=[pl.BlockSpec((BLK, HID_ENC), row),
                  pl.BlockSpec((HID_ENC, HID_ENC), full)],
        out_specs=pl.BlockSpec((BLK, HID_ENC), row),
        out_shape=jax.ShapeDtypeStruct((N, HID_ENC), jnp.float32),
    )(h1, params["W_t"])


# speedup vs baseline: 2.6856x; 1.0485x over previous
import functools
import jax
import jax.numpy as jnp
from jax import lax
from jax.experimental import pallas as pl
from jax.experimental.pallas import tpu as pltpu
from jax.experimental.pallas import tpu_sc as plsc

N_NODES = 10000
N_EDGES = 40000
N_ATOMS = 5
T = 8
B = N_NODES // N_ATOMS
IN_NODE_NF = 16
HID_ENC = 64
TIME_EMB_DIM = 32
HID_DEC = 96
DELTA_FRAME = 1.0
XPAD = 16
MSG_ENC = 128
MSG_DEC = 128
BLK = 2000
BLK_E = 2048
EP = 40960
NROW = 10112
NW = 32
GB = 128
CS = 256
CATW = 256


def _silu(z):
    return z * jax.nn.sigmoid(z)


def _mm(a, b):
    return jax.lax.dot_general(a, b, (((1,), (0,)), ((), ())),
                               precision=jax.lax.Precision.DEFAULT,
                               preferred_element_type=jnp.float32)


def _edge_body(hid, hi, hj, ef, we1, be1, we2, be2, wx1, bx1, wx2, bx2,
               out):
    a = hi[...]
    b = hj[...]
    x = a[:, hid:hid + XPAD] - b[:, hid:hid + XPAD]
    d2 = jnp.sum(x * x, axis=1, keepdims=True)
    blk = a.shape[0]
    catpad = jnp.zeros((blk, CATW - 2 * hid - 1 - 4), jnp.float32)
    cat = jnp.concatenate([a[:, :hid], b[:, :hid], d2, ef[...], catpad],
                          axis=1)
    m = _silu(_mm(cat, we1[...]) + be1[...])
    m = _silu(_mm(m, we2[...]) + be2[...])
    u = _silu(_mm(m, wx1[...]) + bx1[...])
    w = _mm(u, wx2[...]) + bx2[...]
    blk = a.shape[0]
    ones = jnp.ones((blk, 1), jnp.float32)
    pad = out.shape[1] - (m.shape[1] + 4 + 1)
    out[...] = jnp.concatenate(
        [m, x[:, :4] * w, ones, jnp.zeros((blk, pad), jnp.float32)], axis=1)


def _edge_mlp(hi, hj, ef, p, hid, msg_w):
    E = hi.shape[0]
    we1 = jnp.pad(p["We1"], ((0, CATW - p["We1"].shape[0]), (0, 0)))
    grid = E // BLK_E
    row = lambda i: (i, 0)
    full = lambda *_: (0, 0)
    return pl.pallas_call(
        functools.partial(_edge_body, hid),
        grid=(grid,),
        in_specs=[
            pl.BlockSpec((BLK_E, 128), row),
            pl.BlockSpec((BLK_E, 128), row),
            pl.BlockSpec((BLK_E, 4), row),
            pl.BlockSpec((CATW, hid), full),
            pl.BlockSpec((1, hid), full),
            pl.BlockSpec((hid, hid), full),
            pl.BlockSpec((1, hid), full),
            pl.BlockSpec((hid, hid), full),
            pl.BlockSpec((1, hid), full),
            pl.BlockSpec((hid, 1), full),
            pl.BlockSpec((1, 1), full),
        ],
        out_specs=pl.BlockSpec((BLK_E, msg_w), row),
        out_shape=jax.ShapeDtypeStruct((E, msg_w), jnp.float32),
    )(hi, hj, ef, we1, p["be1"][None], p["We2"], p["be2"][None],
      p["Wx1"], p["bx1"][None], p["Wx2"], p["bx2"][None])


def _node_body(emit_tab, hid, x4, h, agg, wh1, bh1, wh2, bh2,
               x4o, ho, tabo):
    a = agg[...]
    agg_h = a[:, :hid]
    cnt = jnp.maximum(a[:, hid + 4:hid + 5], 1.0)
    blk = a.shape[0]
    xupd = jnp.concatenate(
        [a[:, hid:hid + 4], jnp.zeros((blk, XPAD - 4), jnp.float32)], axis=1)
    x4n = x4[...] + xupd / cnt
    x4o[...] = x4n
    hh = h[...]
    cat = jnp.concatenate([hh, agg_h], axis=1)
    upd = _silu(_mm(cat, wh1[...]) + bh1[...])
    hn = hh + _mm(upd, wh2[...]) + bh2[...]
    ho[...] = hn
    if emit_tab:
        pad = jnp.zeros((blk, 128 - hn.shape[1] - XPAD), jnp.float32)
        tabo[...] = jnp.concatenate([hn, x4n, pad], axis=1)


def _node_update(x4, h, agg, p, hid, msg_w, emit_tab=False):
    M = h.shape[0]
    grid = M // BLK
    row = lambda i: (i, 0)
    full = lambda *_: (0, 0)
    outs = [jax.ShapeDtypeStruct((M, XPAD), jnp.float32),
            jax.ShapeDtypeStruct((M, hid), jnp.float32),
            jax.ShapeDtypeStruct((M, 128), jnp.float32)]
    res = pl.pallas_call(
        functools.partial(_node_body, emit_tab, hid),
        grid=(grid,),
        in_specs=[
            pl.BlockSpec((BLK, XPAD), row),
            pl.BlockSpec((BLK, hid), row),
            pl.BlockSpec((BLK, msg_w), row),
            pl.BlockSpec((2 * hid, hid), full),
            pl.BlockSpec((1, hid), full),
            pl.BlockSpec((hid, hid), full),
            pl.BlockSpec((1, hid), full),
        ],
        out_specs=[pl.BlockSpec((BLK, XPAD), row),
                   pl.BlockSpec((BLK, hid), row),
                   pl.BlockSpec((BLK, 128), row)],
        out_shape=outs,
    )(x4, h, agg, p["Wh1"], p["bh1"][None], p["Wh2"], p["bh2"][None])
    if emit_tab:
        return res[0], res[1], res[2]
    return res[0], res[1]


def _embed_body(h0, x4, wemb, bemb, ho, tabo):
    hh = _mm(h0[...], wemb[...]) + bemb[...]
    xx = x4[...]
    ho[...] = hh
    pad = jnp.zeros((hh.shape[0], 128 - hh.shape[1] - XPAD), jnp.float32)
    tabo[...] = jnp.concatenate([hh, xx, pad], axis=1)


def _embed(h0, x4, params):
    N = h0.shape[0]
    hid = HID_ENC
    grid = N // BLK
    row = lambda i: (i, 0)
    full = lambda *_: (0, 0)
    return pl.pallas_call(
        _embed_body,
        grid=(grid,),
        in_specs=[
            pl.BlockSpec((BLK, IN_NODE_NF), row),
            pl.BlockSpec((BLK, XPAD), row),
            pl.BlockSpec((IN_NODE_NF, hid), full),
            pl.BlockSpec((1, hid), full),
        ],
        out_specs=[pl.BlockSpec((BLK, hid), row),
                   pl.BlockSpec((BLK, 128), row)],
        out_shape=[jax.ShapeDtypeStruct((N, hid), jnp.float32),
                   jax.ShapeDtypeStruct((N, 128), jnp.float32)],
    )(h0, x4, params["W_emb"], params["b_emb"][None])


def _time_body(wxp, freqs, h, dh, x4, v4, lm, tfc,
               hho, x4o, v4o, tabo):
    w = wxp[...]
    a, bb, c, d = w[:, 0:1], w[:, 1:2], w[:, 2:3], w[:, 3:4]
    tf = tfc[:, :1]
    xtr = x4[...] - lm[...]
    vv = v4[...]
    xo = xtr + tf * (a * xtr + c * vv) + lm[...]
    vo = vv + tf * (bb * xtr + d * vv)
    x4o[...] = xo
    v4o[...] = vo
    ht = h[...] + tf * dh[...]
    args = tf * freqs[...]
    emb = jnp.concatenate([jnp.sin(args), jnp.cos(args)], axis=1)
    hhcat = jnp.concatenate([ht, emb], axis=1)
    hho[...] = hhcat
    pad = jnp.zeros((ht.shape[0], 128 - HID_DEC - XPAD), jnp.float32)
    tabo[...] = jnp.concatenate([hhcat, xo, pad], axis=1)


def _time_expand(h_rep, dh_rep, x4_rep, v4_rep, lm_rep, tf_col, params):
    M = h_rep.shape[0]
    import numpy as np
    half = TIME_EMB_DIM // 2
    scale = np.log(10000.0) / (half - 1)
    freqs = jnp.exp(-scale * jnp.arange(half, dtype=jnp.float32))[None]
    wxp = params["W_x"].reshape(1, 4)
    grid = M // BLK
    row = lambda i: (i, 0)
    full = lambda *_: (0, 0)
    return pl.pallas_call(
        _time_body,
        grid=(grid,),
        in_specs=[
            pl.BlockSpec((1, 4), full),
            pl.BlockSpec((1, half), full),
            pl.BlockSpec((BLK, HID_ENC), row),
            pl.BlockSpec((BLK, HID_ENC), row),
            pl.BlockSpec((BLK, XPAD), row),
            pl.BlockSpec((BLK, XPAD), row),
            pl.BlockSpec((BLK, XPAD), row),
            pl.BlockSpec((BLK, 8), row),
        ],
        out_specs=[pl.BlockSpec((BLK, HID_DEC), row),
                   pl.BlockSpec((BLK, XPAD), row),
                   pl.BlockSpec((BLK, XPAD), row),
                   pl.BlockSpec((BLK, 128), row)],
        out_shape=[jax.ShapeDtypeStruct((M, HID_DEC), jnp.float32),
                   jax.ShapeDtypeStruct((M, XPAD), jnp.float32),
                   jax.ShapeDtypeStruct((M, XPAD), jnp.float32),
                   jax.ShapeDtypeStruct((M, 128), jnp.float32)],
    )(wxp, freqs, h_rep, dh_rep, x4_rep, v4_rep, lm_rep, tf_col)


def _sc_gather(tab, idx0, idx1, E, D, CS=CS):
    C = E // (NW * CS)
    R = E // (NW * GB)
    K = CS // GB
    assert C % 2 == 0
    mesh = plsc.VectorSubcoreMesh(core_axis_name="c", subcore_axis_name="s",
                                  num_cores=2, num_subcores=16)

    def body(tab_h, i0_h, i1_h, hi_h, hj_h, iv0, iv1, b0, b1, gs0, gs1,
             os0, os1):
        w = lax.axis_index("s") * 2 + lax.axis_index("c")
        pltpu.sync_copy(i0_h.at[w], iv0)
        pltpu.sync_copy(i1_h.at[w], iv1)

        def run(iv, out):
            base = w * C * CS

            def fire(c, buf, gs):
                for j in range(K):
                    pltpu.async_copy(tab_h.at[iv.at[c * K + j]],
                                     buf.at[pl.ds(j * GB, GB)], gs)

            def pair(i, _):
                c0 = 2 * i
                c1 = 2 * i + 1

                @pl.when(i > 0)
                def _():
                    pltpu.make_async_copy(
                        b0, out.at[pl.ds(base, CS)], os0).wait()

                fire(c0, b0, gs0)

                @pl.when(i > 0)
                def _():
                    pltpu.make_async_copy(
                        b1, out.at[pl.ds(base, CS)], os1).wait()

                fire(c1, b1, gs1)
                for j in range(K):
                    pltpu.make_async_copy(
                        tab_h.at[iv.at[j]], b0.at[pl.ds(j * GB, GB)],
                        gs0).wait()
                pltpu.async_copy(b0, out.at[pl.ds(base + c0 * CS, CS)], os0)
                for j in range(K):
                    pltpu.make_async_copy(
                        tab_h.at[iv.at[j]], b1.at[pl.ds(j * GB, GB)],
                        gs1).wait()
                pltpu.async_copy(b1, out.at[pl.ds(base + c1 * CS, CS)], os1)
                return 0

            lax.fori_loop(0, C // 2, pair, 0)
            pltpu.make_async_copy(b0, out.at[pl.ds(base, CS)], os0).wait()
            pltpu.make_async_copy(b1, out.at[pl.ds(base, CS)], os1).wait()

        run(iv0, hi_h)
        run(iv1, hj_h)

    f = pl.kernel(
        body,
        out_type=[jax.ShapeDtypeStruct((E, D), jnp.float32),
                  jax.ShapeDtypeStruct((E, D), jnp.float32)],
        mesh=mesh,
        scratch_types=[pltpu.VMEM((R, GB), jnp.int32),
                       pltpu.VMEM((R, GB), jnp.int32),
                       pltpu.VMEM((CS, D), jnp.float32),
                       pltpu.VMEM((CS, D), jnp.float32),
                       pltpu.SemaphoreType.DMA,
                       pltpu.SemaphoreType.DMA,
                       pltpu.SemaphoreType.DMA,
                       pltpu.SemaphoreType.DMA],
    )
    return f(tab, idx0, idx1)


def _sc_scatter(msg, idx, zrows, nt, D):
    SCS = 128
    ET = EP // 16
    CT = ET // SCS
    K = SCS // GB
    RT = EP // GB // 16
    WR = NROW // 16
    mesh = plsc.VectorSubcoreMesh(core_axis_name="c", subcore_axis_name="s", num_cores=2, num_subcores=16)

    def body(msg_h, idx_h, z_h, agg_h, iv, mbuf, acc):
        cc = lax.axis_index("c")
        s = lax.axis_index("s")
        pltpu.sync_copy(idx_h.at[s], iv)
        ncopies = msg_h.shape[0] // EP
        for k in range(nt):
            t = cc * nt + k

            @pl.when(t < ncopies)
            def _per_copy():
                pltpu.sync_copy(z_h, acc.at[pl.ds(s * WR, WR)])
                plsc.subcore_barrier()

                def chunk(c, _):
                    base = t * EP + s * ET + c * SCS
                    pltpu.sync_copy(msg_h.at[pl.ds(base, SCS)], mbuf)
                    for j in range(K):
                        pltpu.sync_copy(mbuf.at[pl.ds(j * GB, GB)],
                                        acc.at[iv.at[c * K + j]], add=True)
                    return 0

                lax.fori_loop(0, CT, chunk, 0)
                plsc.subcore_barrier()

                @pl.when(s < 15)
                def _():
                    pltpu.sync_copy(acc.at[pl.ds(s * WR, WR)],
                                    agg_h.at[pl.ds(t * N_NODES + s * WR, WR)])

                @pl.when(s == 15)
                def _():
                    pltpu.sync_copy(
                        acc.at[pl.ds(15 * WR, N_NODES - 15 * WR)],
                        agg_h.at[pl.ds(t * N_NODES + 15 * WR,
                                       N_NODES - 15 * WR)])

    ncopies = msg.shape[0] // EP
    f = pl.kernel(
        body,
        out_type=jax.ShapeDtypeStruct((ncopies * N_NODES, D), jnp.float32),
        mesh=mesh,
        scratch_types=[pltpu.VMEM((RT, GB), jnp.int32),
                       pltpu.VMEM((SCS, D), jnp.float32),
                       pltpu.VMEM_SHARED((NROW, D), jnp.float32)],
    )
    return f(msg, idx, zrows)


def _pad4(x):
    return jnp.pad(x, ((0, 0), (0, XPAD - x.shape[1])))


def kernel(x, h, edge_index, edge_fea, v, loc_mean, timeframes, params):
    e0 = edge_index[0]
    e1 = edge_index[1]
    npad = EP - N_EDGES

    e0p = jnp.concatenate([e0, jnp.zeros((npad,), e0.dtype)])
    e1p = jnp.concatenate([e1, jnp.zeros((npad,), e1.dtype)])
    scat_idx = jnp.concatenate(
        [e0, jnp.full((npad,), N_NODES, e0.dtype)]).reshape(16, EP // GB // 16, GB)
    offs = jnp.repeat(jnp.arange(T, dtype=e0.dtype) * N_NODES, EP)
    rdec = T * EP // (NW * GB)
    renc = EP // (NW * GB)
    idx0_dec = (jnp.tile(e0p, (T,)) + offs).reshape(NW, rdec, GB)
    idx1_dec = (jnp.tile(e1p, (T,)) + offs).reshape(NW, rdec, GB)
    idx0_enc = e0p.reshape(NW, renc, GB)
    idx1_enc = e1p.reshape(NW, renc, GB)
    efp = jnp.concatenate([edge_fea, jnp.zeros((npad, 4), jnp.float32)])
    ef_dec = jnp.tile(efp, (T, 1))
    zrows_enc = jnp.zeros((NROW // 16, MSG_ENC), jnp.float32)
    zrows_dec = jnp.zeros((NROW // 16, MSG_DEC), jnp.float32)

    x4 = _pad4(x)
    h1, tab = _embed(h, x4, params)
    for i in range(2):
        p = params["enc%d" % i]
        hi, hj = _sc_gather(tab, idx0_enc, idx1_enc, EP, 128, 128)
        msg = _edge_mlp(hi, hj, efp, p, HID_ENC, MSG_ENC)
        agg = _sc_scatter(msg, scat_idx, zrows_enc, 1, MSG_ENC)
        if i == 0:
            x4, h1, tab = _node_update(x4, h1, agg, p, HID_ENC, MSG_ENC, True)
        else:
            x4, h1 = _node_update(x4, h1, agg, p, HID_ENC, MSG_ENC)

    dh = _dh_only(h1, params)

    tf_t = (timeframes / DELTA_FRAME).T
    tf_node = jnp.repeat(tf_t.reshape(-1), N_ATOMS)
    tf_col = jnp.broadcast_to(tf_node[:, None], (T * N_NODES, 8))
    tile = lambda a: jnp.tile(a, (T, 1))
    hh, x4d, v4d, tab = _time_expand(
        tile(h1), tile(dh), tile(x4), tile(_pad4(v)), tile(_pad4(loc_mean)),
        tf_col, params)

    for i in range(2):
        p = params["dec%d" % i]
        hi, hj = _sc_gather(tab, idx0_dec, idx1_dec, T * EP, 128)
        msg = _edge_mlp(hi, hj, ef_dec, p, HID_DEC, MSG_DEC)
        agg = _sc_scatter(msg, scat_idx, zrows_dec, 4, MSG_DEC)
        if i == 0:
            x4d, hh, tab = _node_update(x4d, hh, agg, p, HID_DEC, MSG_DEC,
                                        True)
        else:
            x4d, hh = _node_update(x4d, hh, agg, p, HID_DEC, MSG_DEC)

    xx = x4d[:, :3]
    vv = v4d[:, :3]
    kld = jnp.asarray(0.0, jnp.float32)
    return (xx, vv, hh, kld)


def _dh_body(h, wt, dho):
    dho[...] = _silu(_mm(h[...], wt[...]))


def _dh_only(h1, params):
    N = h1.shape[0]
    grid = N // BLK
    row = lambda i: (i, 0)
    full = lambda *_: (0, 0)
    return pl.pallas_call(
        _dh_body,
        grid=(grid,),
        in_specs=[pl.BlockSpec((BLK, HID_ENC), row),
                  pl.BlockSpec((HID_ENC, HID_ENC), full)],
        out_specs=pl.BlockSpec((BLK, HID_ENC), row),
        out_shape=jax.ShapeDtypeStruct((N, HID_ENC), jnp.float32),
    )(h1, params["W_t"])


# speedup vs baseline: 2.7553x; 1.0259x over previous
import functools
import jax
import jax.numpy as jnp
from jax import lax
from jax.experimental import pallas as pl
from jax.experimental.pallas import tpu as pltpu
from jax.experimental.pallas import tpu_sc as plsc

N_NODES = 10000
N_EDGES = 40000
N_ATOMS = 5
T = 8
B = N_NODES // N_ATOMS
IN_NODE_NF = 16
HID_ENC = 64
TIME_EMB_DIM = 32
HID_DEC = 96
DELTA_FRAME = 1.0
XPAD = 16
MSG_ENC = 128
MSG_DEC = 128
BLK = 2000
BLK_E = 2048
EP = 40960
NROW = 10112
NW = 32
GB = 128
CS = 256
CATW = 256


def _silu(z):
    return z * jax.nn.sigmoid(z)


def _mm(a, b):
    return jax.lax.dot_general(a, b, (((1,), (0,)), ((), ())),
                               precision=jax.lax.Precision.DEFAULT,
                               preferred_element_type=jnp.float32)


def _edge_body(hid, hi, hj, ef, we1, be1, we2, be2, wx1, bx1, wx2, bx2,
               out):
    a = hi[...]
    b = hj[...]
    x = a[:, hid:hid + XPAD] - b[:, hid:hid + XPAD]
    d2 = jnp.sum(x * x, axis=1, keepdims=True)
    blk = a.shape[0]
    catpad = jnp.zeros((blk, CATW - 2 * hid - 1 - 4), jnp.float32)
    cat = jnp.concatenate([a[:, :hid], b[:, :hid], d2, ef[...], catpad],
                          axis=1)
    m = _silu(_mm(cat, we1[...]) + be1[...])
    m = _silu(_mm(m, we2[...]) + be2[...])
    u = _silu(_mm(m, wx1[...]) + bx1[...])
    w = _mm(u, wx2[...]) + bx2[...]
    blk = a.shape[0]
    ones = jnp.ones((blk, 1), jnp.float32)
    pad = out.shape[1] - (m.shape[1] + 4 + 1)
    out[...] = jnp.concatenate(
        [m, x[:, :4] * w, ones, jnp.zeros((blk, pad), jnp.float32)], axis=1)


def _edge_mlp(hi, hj, ef, p, hid, msg_w):
    E = hi.shape[0]
    we1 = jnp.pad(p["We1"], ((0, CATW - p["We1"].shape[0]), (0, 0)))
    grid = E // BLK_E
    row = lambda i: (i, 0)
    full = lambda *_: (0, 0)
    return pl.pallas_call(
        functools.partial(_edge_body, hid),
        grid=(grid,),
        in_specs=[
            pl.BlockSpec((BLK_E, 128), row),
            pl.BlockSpec((BLK_E, 128), row),
            pl.BlockSpec((BLK_E, 4), row),
            pl.BlockSpec((CATW, hid), full),
            pl.BlockSpec((1, hid), full),
            pl.BlockSpec((hid, hid), full),
            pl.BlockSpec((1, hid), full),
            pl.BlockSpec((hid, hid), full),
            pl.BlockSpec((1, hid), full),
            pl.BlockSpec((hid, 1), full),
            pl.BlockSpec((1, 1), full),
        ],
        out_specs=pl.BlockSpec((BLK_E, msg_w), row),
        out_shape=jax.ShapeDtypeStruct((E, msg_w), jnp.float32),
    )(hi, hj, ef, we1, p["be1"][None], p["We2"], p["be2"][None],
      p["Wx1"], p["bx1"][None], p["Wx2"], p["bx2"][None])


def _node_body(emit_tab, hid, x4, h, agg, wh1, bh1, wh2, bh2,
               x4o, ho, tabo):
    a = agg[...]
    agg_h = a[:, :hid]
    cnt = jnp.maximum(a[:, hid + 4:hid + 5], 1.0)
    blk = a.shape[0]
    xupd = jnp.concatenate(
        [a[:, hid:hid + 4], jnp.zeros((blk, XPAD - 4), jnp.float32)], axis=1)
    x4n = x4[...] + xupd / cnt
    x4o[...] = x4n
    hh = h[...]
    cat = jnp.concatenate([hh, agg_h], axis=1)
    upd = _silu(_mm(cat, wh1[...]) + bh1[...])
    hn = hh + _mm(upd, wh2[...]) + bh2[...]
    ho[...] = hn
    if emit_tab:
        pad = jnp.zeros((blk, 128 - hn.shape[1] - XPAD), jnp.float32)
        tabo[...] = jnp.concatenate([hn, x4n, pad], axis=1)


def _node_update(x4, h, agg, p, hid, msg_w, emit_tab=False):
    M = h.shape[0]
    grid = M // BLK
    row = lambda i: (i, 0)
    full = lambda *_: (0, 0)
    outs = [jax.ShapeDtypeStruct((M, XPAD), jnp.float32),
            jax.ShapeDtypeStruct((M, hid), jnp.float32),
            jax.ShapeDtypeStruct((M, 128), jnp.float32)]
    res = pl.pallas_call(
        functools.partial(_node_body, emit_tab, hid),
        grid=(grid,),
        in_specs=[
            pl.BlockSpec((BLK, XPAD), row),
            pl.BlockSpec((BLK, hid), row),
            pl.BlockSpec((BLK, msg_w), row),
            pl.BlockSpec((2 * hid, hid), full),
            pl.BlockSpec((1, hid), full),
            pl.BlockSpec((hid, hid), full),
            pl.BlockSpec((1, hid), full),
        ],
        out_specs=[pl.BlockSpec((BLK, XPAD), row),
                   pl.BlockSpec((BLK, hid), row),
                   pl.BlockSpec((BLK, 128), row)],
        out_shape=outs,
    )(x4, h, agg, p["Wh1"], p["bh1"][None], p["Wh2"], p["bh2"][None])
    if emit_tab:
        return res[0], res[1], res[2]
    return res[0], res[1]


def _embed_body(h0, x4, wemb, bemb, ho, tabo):
    hh = _mm(h0[...], wemb[...]) + bemb[...]
    xx = x4[...]
    ho[...] = hh
    pad = jnp.zeros((hh.shape[0], 128 - hh.shape[1] - XPAD), jnp.float32)
    tabo[...] = jnp.concatenate([hh, xx, pad], axis=1)


def _embed(h0, x4, params):
    N = h0.shape[0]
    hid = HID_ENC
    grid = N // BLK
    row = lambda i: (i, 0)
    full = lambda *_: (0, 0)
    return pl.pallas_call(
        _embed_body,
        grid=(grid,),
        in_specs=[
            pl.BlockSpec((BLK, IN_NODE_NF), row),
            pl.BlockSpec((BLK, XPAD), row),
            pl.BlockSpec((IN_NODE_NF, hid), full),
            pl.BlockSpec((1, hid), full),
        ],
        out_specs=[pl.BlockSpec((BLK, hid), row),
                   pl.BlockSpec((BLK, 128), row)],
        out_shape=[jax.ShapeDtypeStruct((N, hid), jnp.float32),
                   jax.ShapeDtypeStruct((N, 128), jnp.float32)],
    )(h0, x4, params["W_emb"], params["b_emb"][None])


def _time_body(wxp, freqs, h, dh, x4, v4, lm, tfc,
               hho, x4o, v4o, tabo):
    w = wxp[...]
    a, bb, c, d = w[:, 0:1], w[:, 1:2], w[:, 2:3], w[:, 3:4]
    tf = tfc[:, :1]
    xtr = x4[...] - lm[...]
    vv = v4[...]
    xo = xtr + tf * (a * xtr + c * vv) + lm[...]
    vo = vv + tf * (bb * xtr + d * vv)
    x4o[...] = xo
    v4o[...] = vo
    ht = h[...] + tf * dh[...]
    args = tf * freqs[...]
    emb = jnp.concatenate([jnp.sin(args), jnp.cos(args)], axis=1)
    hhcat = jnp.concatenate([ht, emb], axis=1)
    hho[...] = hhcat
    pad = jnp.zeros((ht.shape[0], 128 - HID_DEC - XPAD), jnp.float32)
    tabo[...] = jnp.concatenate([hhcat, xo, pad], axis=1)


def _time_expand(h_rep, dh_rep, x4_rep, v4_rep, lm_rep, tf_col, params):
    M = h_rep.shape[0]
    import numpy as np
    half = TIME_EMB_DIM // 2
    scale = np.log(10000.0) / (half - 1)
    freqs = jnp.exp(-scale * jnp.arange(half, dtype=jnp.float32))[None]
    wxp = params["W_x"].reshape(1, 4)
    grid = M // BLK
    row = lambda i: (i, 0)
    full = lambda *_: (0, 0)
    return pl.pallas_call(
        _time_body,
        grid=(grid,),
        in_specs=[
            pl.BlockSpec((1, 4), full),
            pl.BlockSpec((1, half), full),
            pl.BlockSpec((BLK, HID_ENC), row),
            pl.BlockSpec((BLK, HID_ENC), row),
            pl.BlockSpec((BLK, XPAD), row),
            pl.BlockSpec((BLK, XPAD), row),
            pl.BlockSpec((BLK, XPAD), row),
            pl.BlockSpec((BLK, 8), row),
        ],
        out_specs=[pl.BlockSpec((BLK, HID_DEC), row),
                   pl.BlockSpec((BLK, XPAD), row),
                   pl.BlockSpec((BLK, XPAD), row),
                   pl.BlockSpec((BLK, 128), row)],
        out_shape=[jax.ShapeDtypeStruct((M, HID_DEC), jnp.float32),
                   jax.ShapeDtypeStruct((M, XPAD), jnp.float32),
                   jax.ShapeDtypeStruct((M, XPAD), jnp.float32),
                   jax.ShapeDtypeStruct((M, 128), jnp.float32)],
    )(wxp, freqs, h_rep, dh_rep, x4_rep, v4_rep, lm_rep, tf_col)


def _sc_gather(tab, idx0, idx1, E, D, CS=128):
    C = E // (NW * CS)
    R = E // (NW * GB)
    assert CS == GB and C % 2 == 0
    mesh = plsc.VectorSubcoreMesh(core_axis_name="c", subcore_axis_name="s",
                                  num_cores=2, num_subcores=16)

    def body(tab_h, i0_h, i1_h, hi_h, hj_h, iv0, iv1,
             ba0, ba1, bb0, bb1, ga0, ga1, gb0, gb1, oa0, oa1, ob0, ob1):
        w = lax.axis_index("s") * 2 + lax.axis_index("c")
        pltpu.sync_copy(i0_h.at[w], iv0)
        pltpu.sync_copy(i1_h.at[w], iv1)
        base = w * C * CS
        units = [(iv0, hi_h, ba0, ga0, oa0, 0), (iv0, hi_h, ba1, ga1, oa1, 1),
                 (iv1, hj_h, bb0, gb0, ob0, 0), (iv1, hj_h, bb1, gb1, ob1, 1)]

        def pair(i, _):
            for iv, out, buf, gs, osem, par in units:
                @pl.when(i > 0)
                def _():
                    pltpu.make_async_copy(
                        buf, out.at[pl.ds(base, CS)], osem).wait()
                pltpu.async_copy(tab_h.at[iv.at[2 * i + par]], buf, gs)
            for iv, out, buf, gs, osem, par in units:
                pltpu.make_async_copy(tab_h.at[iv.at[0]], buf, gs).wait()
                pltpu.async_copy(
                    buf, out.at[pl.ds(base + (2 * i + par) * CS, CS)], osem)
            return 0

        lax.fori_loop(0, C // 2, pair, 0)
        for iv, out, buf, gs, osem, par in units:
            pltpu.make_async_copy(buf, out.at[pl.ds(base, CS)], osem).wait()

    f = pl.kernel(
        body,
        out_type=[jax.ShapeDtypeStruct((E, D), jnp.float32),
                  jax.ShapeDtypeStruct((E, D), jnp.float32)],
        mesh=mesh,
        scratch_types=[pltpu.VMEM((R, GB), jnp.int32),
                       pltpu.VMEM((R, GB), jnp.int32),
                       pltpu.VMEM((CS, D), jnp.float32),
                       pltpu.VMEM((CS, D), jnp.float32),
                       pltpu.VMEM((CS, D), jnp.float32),
                       pltpu.VMEM((CS, D), jnp.float32),
                       pltpu.SemaphoreType.DMA, pltpu.SemaphoreType.DMA,
                       pltpu.SemaphoreType.DMA, pltpu.SemaphoreType.DMA,
                       pltpu.SemaphoreType.DMA, pltpu.SemaphoreType.DMA,
                       pltpu.SemaphoreType.DMA, pltpu.SemaphoreType.DMA],
    )
    return f(tab, idx0, idx1)


def _sc_scatter(msg, idx, zrows, nt, D):
    SCS = 128
    ET = EP // 16
    CT = ET // SCS
    K = SCS // GB
    RT = EP // GB // 16
    WR = NROW // 16
    mesh = plsc.VectorSubcoreMesh(core_axis_name="c", subcore_axis_name="s", num_cores=2, num_subcores=16)

    def body(msg_h, idx_h, z_h, agg_h, iv, mbuf, acc):
        cc = lax.axis_index("c")
        s = lax.axis_index("s")
        pltpu.sync_copy(idx_h.at[s], iv)
        ncopies = msg_h.shape[0] // EP
        for k in range(nt):
            t = cc * nt + k

            @pl.when(t < ncopies)
            def _per_copy():
                pltpu.sync_copy(z_h, acc.at[pl.ds(s * WR, WR)])
                plsc.subcore_barrier()

                def chunk(c, _):
                    base = t * EP + s * ET + c * SCS
                    pltpu.sync_copy(msg_h.at[pl.ds(base, SCS)], mbuf)
                    for j in range(K):
                        pltpu.sync_copy(mbuf.at[pl.ds(j * GB, GB)],
                                        acc.at[iv.at[c * K + j]], add=True)
                    return 0

                lax.fori_loop(0, CT, chunk, 0)
                plsc.subcore_barrier()

                @pl.when(s < 15)
                def _():
                    pltpu.sync_copy(acc.at[pl.ds(s * WR, WR)],
                                    agg_h.at[pl.ds(t * N_NODES + s * WR, WR)])

                @pl.when(s == 15)
                def _():
                    pltpu.sync_copy(
                        acc.at[pl.ds(15 * WR, N_NODES - 15 * WR)],
                        agg_h.at[pl.ds(t * N_NODES + 15 * WR,
                                       N_NODES - 15 * WR)])

    ncopies = msg.shape[0] // EP
    f = pl.kernel(
        body,
        out_type=jax.ShapeDtypeStruct((ncopies * N_NODES, D), jnp.float32),
        mesh=mesh,
        scratch_types=[pltpu.VMEM((RT, GB), jnp.int32),
                       pltpu.VMEM((SCS, D), jnp.float32),
                       pltpu.VMEM_SHARED((NROW, D), jnp.float32)],
    )
    return f(msg, idx, zrows)


def _pad4(x):
    return jnp.pad(x, ((0, 0), (0, XPAD - x.shape[1])))


def kernel(x, h, edge_index, edge_fea, v, loc_mean, timeframes, params):
    e0 = edge_index[0]
    e1 = edge_index[1]
    npad = EP - N_EDGES

    e0p = jnp.concatenate([e0, jnp.zeros((npad,), e0.dtype)])
    e1p = jnp.concatenate([e1, jnp.zeros((npad,), e1.dtype)])
    scat_idx = jnp.concatenate(
        [e0, jnp.full((npad,), N_NODES, e0.dtype)]).reshape(16, EP // GB // 16, GB)
    offs = jnp.repeat(jnp.arange(T, dtype=e0.dtype) * N_NODES, EP)
    rdec = T * EP // (NW * GB)
    renc = EP // (NW * GB)
    idx0_dec = (jnp.tile(e0p, (T,)) + offs).reshape(NW, rdec, GB)
    idx1_dec = (jnp.tile(e1p, (T,)) + offs).reshape(NW, rdec, GB)
    idx0_enc = e0p.reshape(NW, renc, GB)
    idx1_enc = e1p.reshape(NW, renc, GB)
    efp = jnp.concatenate([edge_fea, jnp.zeros((npad, 4), jnp.float32)])
    ef_dec = jnp.tile(efp, (T, 1))
    zrows_enc = jnp.zeros((NROW // 16, MSG_ENC), jnp.float32)
    zrows_dec = jnp.zeros((NROW // 16, MSG_DEC), jnp.float32)

    x4 = _pad4(x)
    h1, tab = _embed(h, x4, params)
    for i in range(2):
        p = params["enc%d" % i]
        hi, hj = _sc_gather(tab, idx0_enc, idx1_enc, EP, 128)
        msg = _edge_mlp(hi, hj, efp, p, HID_ENC, MSG_ENC)
        agg = _sc_scatter(msg, scat_idx, zrows_enc, 1, MSG_ENC)
        if i == 0:
            x4, h1, tab = _node_update(x4, h1, agg, p, HID_ENC, MSG_ENC, True)
        else:
            x4, h1 = _node_update(x4, h1, agg, p, HID_ENC, MSG_ENC)

    dh = _dh_only(h1, params)

    tf_t = (timeframes / DELTA_FRAME).T
    tf_node = jnp.repeat(tf_t.reshape(-1), N_ATOMS)
    tf_col = jnp.broadcast_to(tf_node[:, None], (T * N_NODES, 8))
    tile = lambda a: jnp.tile(a, (T, 1))
    hh, x4d, v4d, tab = _time_expand(
        tile(h1), tile(dh), tile(x4), tile(_pad4(v)), tile(_pad4(loc_mean)),
        tf_col, params)

    for i in range(2):
        p = params["dec%d" % i]
        hi, hj = _sc_gather(tab, idx0_dec, idx1_dec, T * EP, 128)
        msg = _edge_mlp(hi, hj, ef_dec, p, HID_DEC, MSG_DEC)
        agg = _sc_scatter(msg, scat_idx, zrows_dec, 4, MSG_DEC)
        if i == 0:
            x4d, hh, tab = _node_update(x4d, hh, agg, p, HID_DEC, MSG_DEC,
                                        True)
        else:
            x4d, hh = _node_update(x4d, hh, agg, p, HID_DEC, MSG_DEC)

    xx = x4d[:, :3]
    vv = v4d[:, :3]
    kld = jnp.asarray(0.0, jnp.float32)
    return (xx, vv, hh, kld)


def _dh_body(h, wt, dho):
    dho[...] = _silu(_mm(h[...], wt[...]))


def _dh_only(h1, params):
    N = h1.shape[0]
    grid = N // BLK
    row = lambda i: (i, 0)
    full = lambda *_: (0, 0)
    return pl.pallas_call(
        _dh_body,
        grid=(grid,),
        in_specs=[pl.BlockSpec((BLK, HID_ENC), row),
                  pl.BlockSpec((HID_ENC, HID_ENC), full)],
        out_specs=pl.BlockSpec((BLK, HID_ENC), row),
        out_shape=jax.ShapeDtypeStruct((N, HID_ENC), jnp.float32),
    )(h1, params["W_t"])
